# baseline (speedup 1.0000x reference)
import functools

import jax
import jax.numpy as jnp
from jax import lax
from jax.experimental import pallas as pl
from jax.experimental.pallas import tpu as pltpu
from jax.experimental.pallas import tpu_sc as plsc

DIM = 64
PDIM = 128
NC, NS = 2, 16
NW = NC * NS
CH = 400
NBUF = 2


@jax.jit
def _embed(x_flat, table_p):
    n = x_flat.shape[0]
    per_w = n // NW
    n_chunks = per_w // CH
    assert n_chunks % NBUF == 0 and n_chunks >= 2 * NBUF
    mesh = plsc.VectorSubcoreMesh(core_axis_name="c", subcore_axis_name="s")

    @functools.partial(
        pl.kernel,
        mesh=mesh,
        compiler_params=pltpu.CompilerParams(use_tc_tiling_on_sc=False),
        out_type=jax.ShapeDtypeStruct((n, PDIM), jnp.float32),
        scratch_types=[
            pltpu.VMEM((per_w,), jnp.int32),
            pltpu.VMEM((NBUF, CH, PDIM), jnp.float32),
            pltpu.SemaphoreType.DMA,
            pltpu.SemaphoreType.DMA,
        ],
    )
    def k(x_hbm, table_hbm, out_hbm, idx_v, rows_v, sem0, sem1):
        sems = (sem0, sem1)
        wid = lax.axis_index("s") * NC + lax.axis_index("c")
        base = wid * per_w

        pltpu.sync_copy(x_hbm.at[pl.ds(base, per_w)], idx_v)

        def gather(b, c):
            pltpu.async_copy(
                table_hbm.at[idx_v.at[pl.ds(c * CH, CH)]], rows_v.at[b], sems[b]
            )

        def wait_gather(b, c):
            pltpu.make_async_copy(
                table_hbm.at[idx_v.at[pl.ds(c * CH, CH)]], rows_v.at[b], sems[b]
            ).wait()

        def store(b, c):
            pltpu.sync_copy(
                rows_v.at[b, :, pl.ds(0, DIM)],
                out_hbm.at[pl.ds(base + c * CH, CH), pl.ds(0, DIM)],
            )

        for b in range(NBUF):
            gather(b, b)

        def pair_body(g, carry):
            for b in range(NBUF):
                c = NBUF * g + b
                wait_gather(b, c)
                store(b, c)
                gather(b, c + NBUF)
            return carry

        lax.fori_loop(0, n_chunks // NBUF - 1, pair_body, 0)

        for b in range(NBUF):
            c = n_chunks - NBUF + b
            wait_gather(b, c)
            store(b, c)

    return k(x_flat, table_p)


def kernel(x, table):
    n = x.shape[0] * x.shape[1]
    k = 4
    v = table.shape[0] // k
    table_p = jnp.concatenate(
        [jnp.pad(table[i * v:(i + 1) * v], ((0, 0), (0, PDIM - DIM)))
         for i in range(k)], axis=0)
    out_p = _embed(x.reshape(n).astype(jnp.int32), table_p)
    return out_p[:, :DIM].reshape(x.shape[0], x.shape[1], DIM)


# speedup vs baseline: 1.7833x; 1.7833x over previous
import functools

import jax
import jax.numpy as jnp
from jax import lax
from jax.experimental import pallas as pl
from jax.experimental.pallas import tpu as pltpu
from jax.experimental.pallas import tpu_sc as plsc

DIM = 64
PDIM = 128
NC, NS = 2, 16
NW = NC * NS
CH = 256
NBUF = 3


@jax.jit
def _embed(x_flat, table_p):
    n = x_flat.shape[0]
    per_w = n // NW
    n_chunks = per_w // CH
    assert (n_chunks - 7) % NBUF == 0 and n_chunks >= 10
    mesh = plsc.VectorSubcoreMesh(core_axis_name="c", subcore_axis_name="s")

    @functools.partial(
        pl.kernel,
        mesh=mesh,
        compiler_params=pltpu.CompilerParams(use_tc_tiling_on_sc=False),
        out_type=jax.ShapeDtypeStruct((n, PDIM), jnp.float32),
        scratch_types=[
            pltpu.VMEM((per_w,), jnp.int32),
            pltpu.VMEM((NBUF, CH, PDIM), jnp.float32),
            pltpu.SemaphoreType.DMA,
            pltpu.SemaphoreType.DMA,
            pltpu.SemaphoreType.DMA,
            pltpu.SemaphoreType.DMA,
            pltpu.SemaphoreType.DMA,
            pltpu.SemaphoreType.DMA,
        ],
    )
    def k(x_hbm, table_hbm, out_hbm, idx_v, rows_v, g0, g1, g2, s0, s1, s2):
        gsems = (g0, g1, g2)
        ssems = (s0, s1, s2)
        wid = lax.axis_index("s") * NC + lax.axis_index("c")
        base = wid * per_w

        pltpu.sync_copy(x_hbm.at[pl.ds(base, per_w)], idx_v)

        def gather(b, c):
            pltpu.async_copy(
                table_hbm.at[idx_v.at[pl.ds(c * CH, CH)]], rows_v.at[b], gsems[b]
            )

        def wait_gather(b, c):
            pltpu.make_async_copy(
                table_hbm.at[idx_v.at[pl.ds(c * CH, CH)]], rows_v.at[b], gsems[b]
            ).wait()

        def store(b, c):
            pltpu.async_copy(
                rows_v.at[b, :, pl.ds(0, DIM)],
                out_hbm.at[pl.ds(base + c * CH, CH), pl.ds(0, DIM)],
                ssems[b],
            )

        def wait_store(b, c):
            pltpu.make_async_copy(
                rows_v.at[b, :, pl.ds(0, DIM)],
                out_hbm.at[pl.ds(base + c * CH, CH), pl.ds(0, DIM)],
                ssems[b],
            ).wait()

        gather(0, 0)
        gather(1, 1)

        wait_gather(0, 0)
        store(0, 0)
        gather(2, 2)
        wait_gather(1, 1)
        store(1, 1)
        wait_store(0, 0)
        gather(0, 3)
        wait_gather(2, 2)
        store(2, 2)
        wait_store(1, 1)
        gather(1, 4)

        def group(g, carry):
            for p in range(NBUF):
                c = NBUF * g + p
                wait_gather(p, c)
                store(p, c)
                wait_store((p + 2) % NBUF, c - 1)
                gather((p + 2) % NBUF, c + 2)
            return carry

        lax.fori_loop(1, (n_chunks - 7) // NBUF + 1, group, 0)

        nc = n_chunks
        wait_gather(0, nc - 4)
        store(0, nc - 4)
        wait_store(2, nc - 5)
        gather(2, nc - 2)
        wait_gather(1, nc - 3)
        store(1, nc - 3)
        wait_store(0, nc - 4)
        gather(0, nc - 1)
        wait_gather(2, nc - 2)
        store(2, nc - 2)
        wait_store(1, nc - 3)
        wait_gather(0, nc - 1)
        store(0, nc - 1)
        wait_store(2, nc - 2)
        wait_store(0, nc - 1)

    return k(x_flat, table_p)


def kernel(x, table):
    n = x.shape[0] * x.shape[1]
    table_p = jnp.pad(table, ((0, 0), (0, PDIM - DIM)))
    out_p = _embed(x.reshape(n).astype(jnp.int32), table_p)
    return out_p[:, :DIM].reshape(x.shape[0], x.shape[1], DIM)


# speedup vs baseline: 1.7865x; 1.0018x over previous
import functools

import jax
import jax.numpy as jnp
from jax import lax
from jax.experimental import pallas as pl
from jax.experimental.pallas import tpu as pltpu
from jax.experimental.pallas import tpu_sc as plsc

DIM = 64
PDIM = 128
NC, NS = 2, 16
NW = NC * NS
CH = 400
NBUF = 2


@jax.jit
def _embed(x_flat, table_p):
    n = x_flat.shape[0]
    per_w = n // NW
    n_chunks = per_w // CH
    assert n_chunks % NBUF == 0 and n_chunks >= 2 * NBUF
    mesh = plsc.VectorSubcoreMesh(core_axis_name="c", subcore_axis_name="s")

    @functools.partial(
        pl.kernel,
        mesh=mesh,
        compiler_params=pltpu.CompilerParams(use_tc_tiling_on_sc=False),
        out_type=jax.ShapeDtypeStruct((n, PDIM), jnp.float32),
        scratch_types=[
            pltpu.VMEM((per_w,), jnp.int32),
            pltpu.VMEM((NBUF, CH, PDIM), jnp.float32),
            pltpu.SemaphoreType.DMA,
            pltpu.SemaphoreType.DMA,
        ],
    )
    def k(x_hbm, table_hbm, out_hbm, idx_v, rows_v, sem0, sem1):
        sems = (sem0, sem1)
        wid = lax.axis_index("s") * NC + lax.axis_index("c")
        base = wid * per_w

        pltpu.sync_copy(x_hbm.at[pl.ds(base, per_w)], idx_v)

        def gather(b, c):
            pltpu.async_copy(
                table_hbm.at[idx_v.at[pl.ds(c * CH, CH)]], rows_v.at[b], sems[b]
            )

        def wait_gather(b, c):
            pltpu.make_async_copy(
                table_hbm.at[idx_v.at[pl.ds(c * CH, CH)]], rows_v.at[b], sems[b]
            ).wait()

        def store(b, c):
            pltpu.sync_copy(
                rows_v.at[b, :, pl.ds(0, DIM)],
                out_hbm.at[pl.ds(base + c * CH, CH), pl.ds(0, DIM)],
            )

        for b in range(NBUF):
            gather(b, b)

        def pair_body(g, carry):
            for b in range(NBUF):
                c = NBUF * g + b
                wait_gather(b, c)
                store(b, c)
                gather(b, c + NBUF)
            return carry

        lax.fori_loop(0, n_chunks // NBUF - 1, pair_body, 0)

        for b in range(NBUF):
            c = n_chunks - NBUF + b
            wait_gather(b, c)
            store(b, c)

    return k(x_flat, table_p)


def kernel(x, table):
    n = x.shape[0] * x.shape[1]
    table_p = jnp.pad(table, ((0, 0), (0, PDIM - DIM)))
    out_p = _embed(x.reshape(n).astype(jnp.int32), table_p)
    return out_p[:, :DIM].reshape(x.shape[0], x.shape[1], DIM)
